# Initial kernel scaffold; baseline (speedup 1.0000x reference)
import jax
import jax.numpy as jnp
from jax.experimental import pallas as pl


def kernel(x, edge_index, corp_x, corp_edge_index, negative_in, W0, b0, W1, b1, bi_weights):
    raise NotImplementedError("write your pallas kernel here")


# speedup vs baseline: 3.6991x; 3.6991x over previous
import jax
import jax.numpy as jnp
from jax import lax
from jax.experimental import pallas as pl
from jax.experimental.pallas import tpu as pltpu
from jax.experimental.pallas import tpu_sc as plsc

N_NODES = 10000
N_EDGES = 320000
D = 128
NEG = 10
INS_LOSS_W = 1e-05

NS = 16
ACC_ROWS = 10240
STRIPE = ACC_ROWS // NS
ZCHUNK = 40
EK = 128
N_CHUNK = N_EDGES // EK


def _sc_segsum_body(y_hbm, src_hbm, dst_hbm, out_hbm,
                    acc_shared, rows_v, src_v, dst_v, zbuf, sem):
    g = lax.axis_index("c")
    s = lax.axis_index("s")

    def _zrow(i, _):
        def _zcol(j, _):
            zbuf[i, pl.ds(j * 16, 16)] = jnp.zeros((16,), jnp.float32)
            return ()
        return lax.fori_loop(0, D // 16, _zcol, ())
    lax.fori_loop(0, ZCHUNK, _zrow, ())
    row0 = s * STRIPE
    def _zcopy(i, _):
        pltpu.sync_copy(zbuf, acc_shared.at[pl.ds(row0 + i * ZCHUNK, ZCHUNK), :])
        return ()
    lax.fori_loop(0, STRIPE // ZCHUNK, _zcopy, ())
    plsc.subcore_barrier()

    ebase = g * N_EDGES
    n_iter = (N_CHUNK - s + NS - 1) // NS
    def _chunk(i, _):
        off = ebase + (s + i * NS) * EK
        pltpu.sync_copy(src_hbm.at[pl.ds(off, EK)], src_v)
        pltpu.sync_copy(dst_hbm.at[pl.ds(off, EK)], dst_v)
        pltpu.async_copy(y_hbm.at[src_v], rows_v, sem).wait()
        pltpu.sync_copy(rows_v, acc_shared.at[dst_v], add=True)
        return ()
    lax.fori_loop(0, n_iter, _chunk, ())
    plsc.subcore_barrier()

    @pl.when(s < NS - 1)
    def _wr_full():
        pltpu.sync_copy(acc_shared.at[pl.ds(row0, STRIPE), :],
                        out_hbm.at[pl.ds(g * N_NODES + row0, STRIPE), :])

    @pl.when(s == NS - 1)
    def _wr_tail():
        tail = N_NODES - (NS - 1) * STRIPE
        base = (NS - 1) * STRIPE
        pltpu.sync_copy(acc_shared.at[pl.ds(base, tail), :],
                        out_hbm.at[pl.ds(g * N_NODES + base, tail), :])


def _sc_segsum(y_flat, src_flat, dst_flat):
    mesh = plsc.VectorSubcoreMesh(core_axis_name="c", subcore_axis_name="s")
    return pl.kernel(
        _sc_segsum_body,
        out_type=jax.ShapeDtypeStruct((2 * N_NODES, D), jnp.float32),
        mesh=mesh,
        scratch_types=[
            pltpu.VMEM_SHARED((ACC_ROWS, D), jnp.float32),
            pltpu.VMEM((EK, D), jnp.float32),
            pltpu.VMEM((EK,), jnp.int32),
            pltpu.VMEM((EK,), jnp.int32),
            pltpu.VMEM((ZCHUNK, D), jnp.float32),
            pltpu.SemaphoreType.DMA,
        ],
    )(y_flat, src_flat, dst_flat)


RB = 1000
NB = (2 * N_NODES) // RB
NB_G0 = N_NODES // RB


def _mm_body(x_ref, w_ref, o_ref):
    o_ref[...] = jnp.dot(x_ref[...], w_ref[...],
                         preferred_element_type=jnp.float32)


def _matmul(x, w):
    return pl.pallas_call(
        _mm_body,
        grid=(NB,),
        in_specs=[pl.BlockSpec((RB, D), lambda i: (i, 0)),
                  pl.BlockSpec((D, D), lambda i: (0, 0))],
        out_specs=pl.BlockSpec((RB, D), lambda i: (i, 0)),
        out_shape=jax.ShapeDtypeStruct((2 * N_NODES, D), jnp.float32),
    )(x, w)


def _relu_mm_body(s_ref, b_ref, w_ref, o_ref):
    h = jnp.maximum(s_ref[...] + b_ref[...], 0.0)
    o_ref[...] = jnp.dot(h, w_ref[...], preferred_element_type=jnp.float32)


def _relu_matmul(s, b, w):
    return pl.pallas_call(
        _relu_mm_body,
        grid=(NB,),
        in_specs=[pl.BlockSpec((RB, D), lambda i: (i, 0)),
                  pl.BlockSpec((1, D), lambda i: (0, 0)),
                  pl.BlockSpec((D, D), lambda i: (0, 0))],
        out_specs=pl.BlockSpec((RB, D), lambda i: (i, 0)),
        out_shape=jax.ShapeDtypeStruct((2 * N_NODES, D), jnp.float32),
    )(s, b.reshape(1, D), w)


def _colsum_body(s_ref, b_ref, o_ref, acc_ref):
    i = pl.program_id(0)

    @pl.when(i == 0)
    def _init():
        acc_ref[...] = jnp.zeros_like(acc_ref)

    h = jnp.maximum(s_ref[...] + b_ref[...], 0.0)
    acc_ref[...] += jnp.sum(h, axis=0, keepdims=True)

    @pl.when(i == pl.num_programs(0) - 1)
    def _fin():
        o_ref[...] = acc_ref[...]


def _colsum_relu(s2_g0, b1):
    return pl.pallas_call(
        _colsum_body,
        grid=(NB_G0,),
        in_specs=[pl.BlockSpec((RB, D), lambda i: (i, 0)),
                  pl.BlockSpec((1, D), lambda i: (0, 0))],
        out_specs=pl.BlockSpec((1, D), lambda i: (0, 0)),
        out_shape=jax.ShapeDtypeStruct((1, D), jnp.float32),
        scratch_shapes=[pltpu.VMEM((1, D), jnp.float32)],
    )(s2_g0, b1.reshape(1, D))


def _bce_pos(z):
    return jnp.maximum(z, 0.0) - z + jnp.log(1.0 + jnp.exp(-jnp.abs(z)))


def _bce_neg(z):
    return jnp.maximum(z, 0.0) + jnp.log(1.0 + jnp.exp(-jnp.abs(z)))


def _loss_body(s_ref, b_ref, cs_ref, bw_ref, neg_ref, o_ref, acc_ref):
    i = pl.program_id(0)

    @pl.when(i == 0)
    def _init():
        acc_ref[0] = 0.0
        acc_ref[1] = 0.0

    c = 1.0 / (1.0 + jnp.exp(-cs_ref[...] / N_NODES))
    u = lax.dot_general(c, bw_ref[...], (((1,), (1,)), ((), ())),
                        preferred_element_type=jnp.float32)

    h = jnp.maximum(s_ref[...] + b_ref[...], 0.0)
    z = jnp.sum(h * u, axis=1)
    is_pos = i < NB_G0
    dgi = jnp.sum(jnp.where(is_pos, _bce_pos(z), _bce_neg(z)))
    acc_ref[0] += dgi

    pos_z = jnp.sum(h * h, axis=1)
    ins = jnp.sum(_bce_pos(pos_z))
    for k in range(NEG):
        nz = jnp.sum(h * neg_ref[:, pl.ds(k * D, D)], axis=1)
        ins = ins + jnp.sum(_bce_neg(nz))
    acc_ref[1] += jnp.where(is_pos, ins, 0.0)

    @pl.when(i == pl.num_programs(0) - 1)
    def _fin():
        o_ref[0, 0] = (acc_ref[0] / (2 * N_NODES)
                       + INS_LOSS_W * acc_ref[1] / N_NODES)


def _loss(s2, b1, colsum, bi_weights, negative_in):
    neg_flat = negative_in.reshape(N_NODES, NEG * D)
    return pl.pallas_call(
        _loss_body,
        grid=(NB,),
        in_specs=[
            pl.BlockSpec((RB, D), lambda i: (i, 0)),
            pl.BlockSpec((1, D), lambda i: (0, 0)),
            pl.BlockSpec((1, D), lambda i: (0, 0)),
            pl.BlockSpec((D, D), lambda i: (0, 0)),
            pl.BlockSpec((RB, NEG * D),
                         lambda i: (jnp.minimum(i, NB_G0 - 1), 0)),
        ],
        out_specs=pl.BlockSpec(memory_space=pltpu.SMEM),
        out_shape=jax.ShapeDtypeStruct((1, 1), jnp.float32),
        scratch_shapes=[pltpu.SMEM((2,), jnp.float32)],
    )(s2, b1.reshape(1, D), colsum, bi_weights, neg_flat)


def kernel(x, edge_index, corp_x, corp_edge_index, negative_in,
           W0, b0, W1, b1, bi_weights):
    x_flat = jnp.concatenate([x, corp_x], axis=0)
    src_flat = jnp.concatenate(
        [edge_index[0], corp_edge_index[0] + N_NODES]).astype(jnp.int32)
    dst_flat = jnp.concatenate(
        [edge_index[1], corp_edge_index[1]]).astype(jnp.int32)

    y0 = _matmul(x_flat, W0)
    s1 = _sc_segsum(y0, src_flat, dst_flat)
    y1 = _relu_matmul(s1, b0, W1)
    s2 = _sc_segsum(y1, src_flat, dst_flat)
    cs = _colsum_relu(s2[:N_NODES], b1)
    out = _loss(s2, b1, cs, bi_weights, negative_in)
    return out.reshape(())


# speedup vs baseline: 5.5798x; 1.5084x over previous
import jax
import jax.numpy as jnp
from jax import lax
from jax.experimental import pallas as pl
from jax.experimental.pallas import tpu as pltpu
from jax.experimental.pallas import tpu_sc as plsc

N_NODES = 10000
N_EDGES = 320000
D = 128
NEG = 10
INS_LOSS_W = 1e-05

NS = 16
ACC_ROWS = 10240
STRIPE = ACC_ROWS // NS
ZCHUNK = 40
EK = 128
IB = 8
IDX_ROWS = 2504
E_PAD = IDX_ROWS * EK
NBLK = IDX_ROWS // IB
PAD_DST = N_NODES + 200


def _sc_segsum_body(y_hbm, src_hbm, dst_hbm, out_hbm,
                    acc_shared, rows_a, rows_b, src_blk, dst_blk, zbuf,
                    gsem_a, gsem_b, ssem_a, ssem_b):
    g = lax.axis_index("c")
    s = lax.axis_index("s")

    def _zrow(i, _):
        def _zcol(j, _):
            zbuf[i, pl.ds(j * 16, 16)] = jnp.zeros((16,), jnp.float32)
            return ()
        return lax.fori_loop(0, D // 16, _zcol, ())
    lax.fori_loop(0, ZCHUNK, _zrow, ())
    row0 = s * STRIPE
    def _zcopy(i, _):
        pltpu.sync_copy(zbuf, acc_shared.at[pl.ds(row0 + i * ZCHUNK, ZCHUNK), :])
        return ()
    lax.fori_loop(0, STRIPE // ZCHUNK, _zcopy, ())
    plsc.subcore_barrier()

    n_iter = (NBLK - s + NS - 1) // NS

    def _block(i, _):
        r0 = g * IDX_ROWS + (s + i * NS) * IB
        pltpu.sync_copy(src_hbm.at[pl.ds(r0, IB), :], src_blk)
        pltpu.sync_copy(dst_hbm.at[pl.ds(r0, IB), :], dst_blk)
        bufs = ((rows_a, gsem_a, ssem_a), (rows_b, gsem_b, ssem_b))
        gd = [None, None]
        sd = [None, None]
        gd[0] = pltpu.async_copy(y_hbm.at[src_blk.at[0]], rows_a, gsem_a)
        for j in range(IB):
            cur = j % 2
            nxt = (j + 1) % 2
            buf, _, ssem = bufs[cur]
            nbuf, ngsem, _ = bufs[nxt]
            if j + 1 < IB:
                if sd[nxt] is not None:
                    sd[nxt].wait()
                gd[nxt] = pltpu.async_copy(
                    y_hbm.at[src_blk.at[j + 1]], nbuf, ngsem)
            gd[cur].wait()
            sd[cur] = pltpu.async_copy(
                buf, acc_shared.at[dst_blk.at[j]], ssem, add=True)
        sd[0].wait()
        sd[1].wait()
        return ()
    lax.fori_loop(0, n_iter, _block, ())
    plsc.subcore_barrier()

    @pl.when(s < NS - 1)
    def _wr_full():
        pltpu.sync_copy(acc_shared.at[pl.ds(row0, STRIPE), :],
                        out_hbm.at[pl.ds(g * N_NODES + row0, STRIPE), :])

    @pl.when(s == NS - 1)
    def _wr_tail():
        tail = N_NODES - (NS - 1) * STRIPE
        base = (NS - 1) * STRIPE
        pltpu.sync_copy(acc_shared.at[pl.ds(base, tail), :],
                        out_hbm.at[pl.ds(g * N_NODES + base, tail), :])


def _sc_segsum(y_flat, src2d, dst2d):
    mesh = plsc.VectorSubcoreMesh(core_axis_name="c", subcore_axis_name="s")
    return pl.kernel(
        _sc_segsum_body,
        out_type=jax.ShapeDtypeStruct((2 * N_NODES, D), jnp.float32),
        mesh=mesh,
        scratch_types=[
            pltpu.VMEM_SHARED((ACC_ROWS, D), jnp.float32),
            pltpu.VMEM((EK, D), jnp.float32),
            pltpu.VMEM((EK, D), jnp.float32),
            pltpu.VMEM((IB, EK), jnp.int32),
            pltpu.VMEM((IB, EK), jnp.int32),
            pltpu.VMEM((ZCHUNK, D), jnp.float32),
            pltpu.SemaphoreType.DMA,
            pltpu.SemaphoreType.DMA,
            pltpu.SemaphoreType.DMA,
            pltpu.SemaphoreType.DMA,
        ],
    )(y_flat, src2d, dst2d)


RB = 1000
NB = (2 * N_NODES) // RB
NB_G0 = N_NODES // RB


def _mm_body(x_ref, w_ref, o_ref):
    o_ref[...] = jnp.dot(x_ref[...], w_ref[...],
                         preferred_element_type=jnp.float32)


def _matmul(x, w):
    return pl.pallas_call(
        _mm_body,
        grid=(NB,),
        in_specs=[pl.BlockSpec((RB, D), lambda i: (i, 0)),
                  pl.BlockSpec((D, D), lambda i: (0, 0))],
        out_specs=pl.BlockSpec((RB, D), lambda i: (i, 0)),
        out_shape=jax.ShapeDtypeStruct((2 * N_NODES, D), jnp.float32),
    )(x, w)


def _relu_mm_body(s_ref, b_ref, w_ref, o_ref):
    h = jnp.maximum(s_ref[...] + b_ref[...], 0.0)
    o_ref[...] = jnp.dot(h, w_ref[...], preferred_element_type=jnp.float32)


def _relu_matmul(s, b, w):
    return pl.pallas_call(
        _relu_mm_body,
        grid=(NB,),
        in_specs=[pl.BlockSpec((RB, D), lambda i: (i, 0)),
                  pl.BlockSpec((1, D), lambda i: (0, 0)),
                  pl.BlockSpec((D, D), lambda i: (0, 0))],
        out_specs=pl.BlockSpec((RB, D), lambda i: (i, 0)),
        out_shape=jax.ShapeDtypeStruct((2 * N_NODES, D), jnp.float32),
    )(s, b.reshape(1, D), w)


def _colsum_body(s_ref, b_ref, o_ref, acc_ref):
    i = pl.program_id(0)

    @pl.when(i == 0)
    def _init():
        acc_ref[...] = jnp.zeros_like(acc_ref)

    h = jnp.maximum(s_ref[...] + b_ref[...], 0.0)
    acc_ref[...] += jnp.sum(h, axis=0, keepdims=True)

    @pl.when(i == pl.num_programs(0) - 1)
    def _fin():
        o_ref[...] = acc_ref[...]


def _colsum_relu(s2_g0, b1):
    return pl.pallas_call(
        _colsum_body,
        grid=(NB_G0,),
        in_specs=[pl.BlockSpec((RB, D), lambda i: (i, 0)),
                  pl.BlockSpec((1, D), lambda i: (0, 0))],
        out_specs=pl.BlockSpec((1, D), lambda i: (0, 0)),
        out_shape=jax.ShapeDtypeStruct((1, D), jnp.float32),
        scratch_shapes=[pltpu.VMEM((1, D), jnp.float32)],
    )(s2_g0, b1.reshape(1, D))


def _bce_pos(z):
    return jnp.maximum(z, 0.0) - z + jnp.log(1.0 + jnp.exp(-jnp.abs(z)))


def _bce_neg(z):
    return jnp.maximum(z, 0.0) + jnp.log(1.0 + jnp.exp(-jnp.abs(z)))


def _loss_body(s_ref, b_ref, cs_ref, bw_ref, neg_ref, o_ref, acc_ref):
    i = pl.program_id(0)

    @pl.when(i == 0)
    def _init():
        acc_ref[0] = 0.0
        acc_ref[1] = 0.0

    c = 1.0 / (1.0 + jnp.exp(-cs_ref[...] / N_NODES))
    u = lax.dot_general(c, bw_ref[...], (((1,), (1,)), ((), ())),
                        preferred_element_type=jnp.float32)

    h = jnp.maximum(s_ref[...] + b_ref[...], 0.0)
    z = jnp.sum(h * u, axis=1)
    is_pos = i < NB_G0
    dgi = jnp.sum(jnp.where(is_pos, _bce_pos(z), _bce_neg(z)))
    acc_ref[0] += dgi

    pos_z = jnp.sum(h * h, axis=1)
    ins = jnp.sum(_bce_pos(pos_z))
    for k in range(NEG):
        nz = jnp.sum(h * neg_ref[:, pl.ds(k * D, D)], axis=1)
        ins = ins + jnp.sum(_bce_neg(nz))
    acc_ref[1] += jnp.where(is_pos, ins, 0.0)

    @pl.when(i == pl.num_programs(0) - 1)
    def _fin():
        o_ref[0, 0] = (acc_ref[0] / (2 * N_NODES)
                       + INS_LOSS_W * acc_ref[1] / N_NODES)


def _loss(s2, b1, colsum, bi_weights, negative_in):
    neg_flat = negative_in.reshape(N_NODES, NEG * D)
    return pl.pallas_call(
        _loss_body,
        grid=(NB,),
        in_specs=[
            pl.BlockSpec((RB, D), lambda i: (i, 0)),
            pl.BlockSpec((1, D), lambda i: (0, 0)),
            pl.BlockSpec((1, D), lambda i: (0, 0)),
            pl.BlockSpec((D, D), lambda i: (0, 0)),
            pl.BlockSpec((RB, NEG * D),
                         lambda i: (jnp.minimum(i, NB_G0 - 1), 0)),
        ],
        out_specs=pl.BlockSpec(memory_space=pltpu.SMEM),
        out_shape=jax.ShapeDtypeStruct((1, 1), jnp.float32),
        scratch_shapes=[pltpu.SMEM((2,), jnp.float32)],
    )(s2, b1.reshape(1, D), colsum, bi_weights, neg_flat)


def kernel(x, edge_index, corp_x, corp_edge_index, negative_in,
           W0, b0, W1, b1, bi_weights):
    x_flat = jnp.concatenate([x, corp_x], axis=0)
    npad = E_PAD - N_EDGES
    src_pad = jnp.zeros((npad,), jnp.int32)
    dst_pad = jnp.full((npad,), PAD_DST, jnp.int32)
    src2d = jnp.concatenate(
        [edge_index[0].astype(jnp.int32), src_pad,
         corp_edge_index[0].astype(jnp.int32) + N_NODES, src_pad]
    ).reshape(2 * IDX_ROWS, EK)
    dst2d = jnp.concatenate(
        [edge_index[1].astype(jnp.int32), dst_pad,
         corp_edge_index[1].astype(jnp.int32), dst_pad]
    ).reshape(2 * IDX_ROWS, EK)

    y0 = _matmul(x_flat, W0)
    s1 = _sc_segsum(y0, src2d, dst2d)
    y1 = _relu_matmul(s1, b0, W1)
    s2 = _sc_segsum(y1, src2d, dst2d)
    cs = _colsum_relu(s2[:N_NODES], b1)
    out = _loss(s2, b1, cs, bi_weights, negative_in)
    return out.reshape(())


# speedup vs baseline: 5.6593x; 1.0142x over previous
import jax
import jax.numpy as jnp
from jax import lax
from jax.experimental import pallas as pl
from jax.experimental.pallas import tpu as pltpu
from jax.experimental.pallas import tpu_sc as plsc

N_NODES = 10000
N_EDGES = 320000
D = 128
NEG = 10
INS_LOSS_W = 1e-05

NC = 2
NS = 16
NW = NC * NS
ACC_ROWS = 10240
STRIPE = ACC_ROWS // NS
ZCHUNK = 40
EK = 128
IB = 8
IDX_ROWS = 2504
E_PAD = IDX_ROWS * EK
NBLK = IDX_ROWS // IB
PAD_DST = N_NODES + 200


def _sc_segsum_body(y_hbm, src_hbm, dst_hbm, out_hbm,
                    acc_shared, rows_a, rows_b, src_blk, dst_blk, zbuf,
                    gsem_a, gsem_b, ssem_a, ssem_b):
    c = lax.axis_index("c")
    s = lax.axis_index("s")
    w = c * NS + s

    def _zrow(i, _):
        def _zcol(j, _):
            zbuf[i, pl.ds(j * 16, 16)] = jnp.zeros((16,), jnp.float32)
            return ()
        return lax.fori_loop(0, D // 16, _zcol, ())
    lax.fori_loop(0, ZCHUNK, _zrow, ())
    row0 = s * STRIPE
    def _zcopy(i, _):
        pltpu.sync_copy(zbuf, acc_shared.at[pl.ds(row0 + i * ZCHUNK, ZCHUNK), :])
        return ()
    lax.fori_loop(0, STRIPE // ZCHUNK, _zcopy, ())
    plsc.subcore_barrier()

    n_iter = (NBLK - w + NW - 1) // NW

    def _block(i, _):
        r0 = (w + i * NW) * IB
        pltpu.sync_copy(src_hbm.at[pl.ds(r0, IB), :], src_blk)
        pltpu.sync_copy(dst_hbm.at[pl.ds(r0, IB), :], dst_blk)
        bufs = ((rows_a, gsem_a, ssem_a), (rows_b, gsem_b, ssem_b))
        gd = [None, None]
        sd = [None, None]
        gd[0] = pltpu.async_copy(y_hbm.at[src_blk.at[0]], rows_a, gsem_a)
        for j in range(IB):
            cur = j % 2
            nxt = (j + 1) % 2
            buf, _, ssem = bufs[cur]
            nbuf, ngsem, _ = bufs[nxt]
            if j + 1 < IB:
                if sd[nxt] is not None:
                    sd[nxt].wait()
                gd[nxt] = pltpu.async_copy(
                    y_hbm.at[src_blk.at[j + 1]], nbuf, ngsem)
            gd[cur].wait()
            sd[cur] = pltpu.async_copy(
                buf, acc_shared.at[dst_blk.at[j]], ssem, add=True)
        sd[0].wait()
        sd[1].wait()
        return ()
    lax.fori_loop(0, n_iter, _block, ())
    plsc.subcore_barrier()

    @pl.when(s < NS - 1)
    def _wr_full():
        pltpu.sync_copy(acc_shared.at[pl.ds(row0, STRIPE), :],
                        out_hbm.at[pl.ds(c * N_NODES + row0, STRIPE), :])

    @pl.when(s == NS - 1)
    def _wr_tail():
        tail = N_NODES - (NS - 1) * STRIPE
        base = (NS - 1) * STRIPE
        pltpu.sync_copy(acc_shared.at[pl.ds(base, tail), :],
                        out_hbm.at[pl.ds(c * N_NODES + base, tail), :])


def _sc_segsum(y, src2d, dst2d):
    mesh = plsc.VectorSubcoreMesh(core_axis_name="c", subcore_axis_name="s")
    return pl.kernel(
        _sc_segsum_body,
        out_type=jax.ShapeDtypeStruct((2 * N_NODES, D), jnp.float32),
        mesh=mesh,
        scratch_types=[
            pltpu.VMEM_SHARED((ACC_ROWS, D), jnp.float32),
            pltpu.VMEM((EK, D), jnp.float32),
            pltpu.VMEM((EK, D), jnp.float32),
            pltpu.VMEM((IB, EK), jnp.int32),
            pltpu.VMEM((IB, EK), jnp.int32),
            pltpu.VMEM((ZCHUNK, D), jnp.float32),
            pltpu.SemaphoreType.DMA,
            pltpu.SemaphoreType.DMA,
            pltpu.SemaphoreType.DMA,
            pltpu.SemaphoreType.DMA,
        ],
    )(y, src2d, dst2d)


RB = 1000
NB_G = N_NODES // RB


def _mm_body(x_ref, w_ref, o_ref):
    o_ref[...] = jnp.dot(x_ref[...], w_ref[...],
                         preferred_element_type=jnp.float32)


def _matmul(x, w):
    return pl.pallas_call(
        _mm_body,
        grid=(NB_G,),
        in_specs=[pl.BlockSpec((RB, D), lambda i: (i, 0)),
                  pl.BlockSpec((D, D), lambda i: (0, 0))],
        out_specs=pl.BlockSpec((RB, D), lambda i: (i, 0)),
        out_shape=jax.ShapeDtypeStruct((N_NODES, D), jnp.float32),
    )(x, w)


def _relu_mm_body(p0_ref, p1_ref, b_ref, w_ref, o_ref):
    h = jnp.maximum(p0_ref[...] + p1_ref[...] + b_ref[...], 0.0)
    o_ref[...] = jnp.dot(h, w_ref[...], preferred_element_type=jnp.float32)


def _relu_matmul(parts, b, w):
    return pl.pallas_call(
        _relu_mm_body,
        grid=(NB_G,),
        in_specs=[pl.BlockSpec((RB, D), lambda i: (i, 0)),
                  pl.BlockSpec((RB, D), lambda i: (i + NB_G, 0)),
                  pl.BlockSpec((1, D), lambda i: (0, 0)),
                  pl.BlockSpec((D, D), lambda i: (0, 0))],
        out_specs=pl.BlockSpec((RB, D), lambda i: (i, 0)),
        out_shape=jax.ShapeDtypeStruct((N_NODES, D), jnp.float32),
    )(parts, parts, b.reshape(1, D), w)


def _colsum_body(p0_ref, p1_ref, b_ref, o_ref, acc_ref):
    i = pl.program_id(0)

    @pl.when(i == 0)
    def _init():
        acc_ref[...] = jnp.zeros_like(acc_ref)

    h = jnp.maximum(p0_ref[...] + p1_ref[...] + b_ref[...], 0.0)
    acc_ref[...] += jnp.sum(h, axis=0, keepdims=True)

    @pl.when(i == pl.num_programs(0) - 1)
    def _fin():
        o_ref[...] = acc_ref[...]


def _colsum_relu(parts, b1):
    return pl.pallas_call(
        _colsum_body,
        grid=(NB_G,),
        in_specs=[pl.BlockSpec((RB, D), lambda i: (i, 0)),
                  pl.BlockSpec((RB, D), lambda i: (i + NB_G, 0)),
                  pl.BlockSpec((1, D), lambda i: (0, 0))],
        out_specs=pl.BlockSpec((1, D), lambda i: (0, 0)),
        out_shape=jax.ShapeDtypeStruct((1, D), jnp.float32),
        scratch_shapes=[pltpu.VMEM((1, D), jnp.float32)],
    )(parts, parts, b1.reshape(1, D))


def _bce_pos(z):
    return jnp.maximum(z, 0.0) - z + jnp.log(1.0 + jnp.exp(-jnp.abs(z)))


def _bce_neg(z):
    return jnp.maximum(z, 0.0) + jnp.log(1.0 + jnp.exp(-jnp.abs(z)))


def _loss_body(pa0_ref, pa1_ref, pb0_ref, pb1_ref, b_ref, cs_ref, bw_ref,
               neg_ref, o_ref, acc_ref):
    i = pl.program_id(0)

    @pl.when(i == 0)
    def _init():
        acc_ref[0] = 0.0
        acc_ref[1] = 0.0

    c = 1.0 / (1.0 + jnp.exp(-cs_ref[...] / N_NODES))
    u = lax.dot_general(c, bw_ref[...], (((1,), (1,)), ((), ())),
                        preferred_element_type=jnp.float32)

    is_pos = i < NB_G
    ha = jnp.maximum(pa0_ref[...] + pa1_ref[...] + b_ref[...], 0.0)
    hb = jnp.maximum(pb0_ref[...] + pb1_ref[...] + b_ref[...], 0.0)
    h = jnp.where(is_pos, ha, hb)
    z = jnp.sum(h * u, axis=1)
    acc_ref[0] += jnp.sum(jnp.where(is_pos, _bce_pos(z), _bce_neg(z)))

    pos_z = jnp.sum(h * h, axis=1)
    negs = jnp.sum(h[:, None, :] * neg_ref[...], axis=2)
    ins = jnp.sum(_bce_pos(pos_z)) + jnp.sum(_bce_neg(negs))
    acc_ref[1] += jnp.where(is_pos, ins, 0.0)

    @pl.when(i == pl.num_programs(0) - 1)
    def _fin():
        o_ref[0, 0] = (acc_ref[0] / (2 * N_NODES)
                       + INS_LOSS_W * acc_ref[1] / N_NODES)


def _loss(s2a, s2b, b1, colsum, bi_weights, negative_in):
    return pl.pallas_call(
        _loss_body,
        grid=(2 * NB_G,),
        in_specs=[
            pl.BlockSpec((RB, D), lambda i: (jnp.minimum(i, NB_G - 1), 0)),
            pl.BlockSpec((RB, D),
                         lambda i: (jnp.minimum(i, NB_G - 1) + NB_G, 0)),
            pl.BlockSpec((RB, D), lambda i: (jnp.maximum(i - NB_G, 0), 0)),
            pl.BlockSpec((RB, D),
                         lambda i: (jnp.maximum(i - NB_G, 0) + NB_G, 0)),
            pl.BlockSpec((1, D), lambda i: (0, 0)),
            pl.BlockSpec((1, D), lambda i: (0, 0)),
            pl.BlockSpec((D, D), lambda i: (0, 0)),
            pl.BlockSpec((RB, NEG, D),
                         lambda i: (jnp.minimum(i, NB_G - 1), 0, 0)),
        ],
        out_specs=pl.BlockSpec(memory_space=pltpu.SMEM),
        out_shape=jax.ShapeDtypeStruct((1, 1), jnp.float32),
        scratch_shapes=[pltpu.SMEM((2,), jnp.float32)],
    )(s2a, s2a, s2b, s2b, b1.reshape(1, D), colsum, bi_weights, negative_in)


def _edges_2d(ei):
    npad = E_PAD - N_EDGES
    src = jnp.concatenate(
        [ei[0].astype(jnp.int32), jnp.zeros((npad,), jnp.int32)]
    ).reshape(IDX_ROWS, EK)
    dst = jnp.concatenate(
        [ei[1].astype(jnp.int32), jnp.full((npad,), PAD_DST, jnp.int32)]
    ).reshape(IDX_ROWS, EK)
    return src, dst


def kernel(x, edge_index, corp_x, corp_edge_index, negative_in,
           W0, b0, W1, b1, bi_weights):
    src_a, dst_a = _edges_2d(edge_index)
    src_b, dst_b = _edges_2d(corp_edge_index)

    y0a = _matmul(x, W0)
    y0b = _matmul(corp_x, W0)
    s1a = _sc_segsum(y0a, src_a, dst_a)
    s1b = _sc_segsum(y0b, src_b, dst_b)
    y1a = _relu_matmul(s1a, b0, W1)
    s2a = _sc_segsum(y1a, src_a, dst_a)
    y1b = _relu_matmul(s1b, b0, W1)
    s2b = _sc_segsum(y1b, src_b, dst_b)
    cs = _colsum_relu(s2a, b1)
    out = _loss(s2a, s2b, b1, cs, bi_weights, negative_in)
    return out.reshape(())
